# speedup vs baseline: 8.9894x; 8.9894x over previous
import jax
import jax.numpy as jnp
from jax.experimental import pallas as pl

_B, _C, _S = 4, 2048, 3
_V, _H = 1000, 128
_ITERS = 2
_VP = 1024
_CBLK = 512
_CB = _C // _CBLK


def _iter_body(idx_ref, vs_ref, wvc_ref, bvc_ref, wce_ref, bce_ref,
               wcv_ref, bcv_ref, msum_ref, cnt_ref):
    b = pl.program_id(0)
    cb = pl.program_id(1)
    step = b * _CB + cb

    @pl.when(step == 0)
    def _zero():
        msum_ref[...] = jnp.zeros_like(msum_ref)
        cnt_ref[...] = jnp.zeros_like(cnt_ref)

    idx = idx_ref[0, 0]
    vio = jax.lax.broadcasted_iota(jnp.int32, (_CBLK, _VP), 1)
    m = jnp.zeros((_CBLK, _VP), jnp.float32)
    for s in range(_S):
        m = m + (idx[s, :][:, None] == vio).astype(jnp.float32)
    occ = (m > 0).astype(jnp.float32)

    vs = vs_ref[0]
    g = jax.lax.dot_general(m, vs, (((1,), (0,)), ((), ())),
                            preferred_element_type=jnp.float32) * (1.0 / _S)
    h = jnp.dot(g, wvc_ref[...], preferred_element_type=jnp.float32) + bvc_ref[...]
    cl = jnp.dot(h, wce_ref[...], preferred_element_type=jnp.float32) + bce_ref[...]
    t = jnp.dot(cl, wcv_ref[...], preferred_element_type=jnp.float32) + bcv_ref[...]

    msum_ref[...] += jax.lax.dot_general(occ, t, (((0,), (0,)), ((), ())),
                                         preferred_element_type=jnp.float32)
    cnt_ref[...] += jnp.sum(occ, axis=0, keepdims=True)


def _iter_call(idx4, vs_p, wvc_t, bvc2, wce_t, bce2, wcv_t, bcv2):
    wspec = pl.BlockSpec((_H, _H), lambda b, cb: (0, 0))
    bspec = pl.BlockSpec((1, _H), lambda b, cb: (0, 0))
    return pl.pallas_call(
        _iter_body,
        grid=(_B, _CB),
        in_specs=[
            pl.BlockSpec((1, 1, _S, _CBLK), lambda b, cb: (b, cb, 0, 0)),
            pl.BlockSpec((1, _VP, _H), lambda b, cb: (b, 0, 0)),
            wspec, bspec, wspec, bspec, wspec, bspec,
        ],
        out_specs=[
            pl.BlockSpec((_VP, _H), lambda b, cb: (0, 0)),
            pl.BlockSpec((1, _VP), lambda b, cb: (0, 0)),
        ],
        out_shape=[
            jax.ShapeDtypeStruct((_VP, _H), jnp.float32),
            jax.ShapeDtypeStruct((1, _VP), jnp.float32),
        ],
    )(idx4, vs_p, wvc_t, bvc2, wce_t, bce2, wcv_t, bcv2)


def _update_body(vs_ref, msum_ref, cnt_ref, out_ref):
    cnt = cnt_ref[0, :]
    scale = (cnt > 0).astype(jnp.float32) / jnp.maximum(cnt, 1.0)
    msgs = msum_ref[...] * scale[:, None]
    out_ref[...] = vs_ref[...] + msgs[None, :, :]


def _update_call(vs_p, msum, cnt):
    return pl.pallas_call(
        _update_body,
        out_shape=jax.ShapeDtypeStruct((_B, _VP, _H), jnp.float32),
    )(vs_p, msum, cnt)


def _head_body(vs_ref, msum_ref, cnt_ref, w1_ref, b1_ref, w2_ref, b2_ref, out_ref):
    cnt = cnt_ref[0, :]
    scale = (cnt > 0).astype(jnp.float32) / jnp.maximum(cnt, 1.0)
    msgs = msum_ref[...] * scale[:, None]
    vsn = vs_ref[0] + msgs
    hh = jnp.maximum(
        jnp.dot(vsn, w1_ref[...], preferred_element_type=jnp.float32) + b1_ref[...],
        0.0)
    logit = jnp.sum(hh * w2_ref[...], axis=1, keepdims=True) + b2_ref[...]
    out_ref[0] = jax.nn.sigmoid(logit)


def _head_call(vs_p, msum, cnt, w1_t, b12, w22, b22):
    return pl.pallas_call(
        _head_body,
        grid=(_B,),
        in_specs=[
            pl.BlockSpec((1, _VP, _H), lambda b: (b, 0, 0)),
            pl.BlockSpec((_VP, _H), lambda b: (0, 0)),
            pl.BlockSpec((1, _VP), lambda b: (0, 0)),
            pl.BlockSpec((_H, _H), lambda b: (0, 0)),
            pl.BlockSpec((1, _H), lambda b: (0, 0)),
            pl.BlockSpec((1, _H), lambda b: (0, 0)),
            pl.BlockSpec((1, 1), lambda b: (0, 0)),
        ],
        out_specs=pl.BlockSpec((1, _VP, 1), lambda b: (b, 0, 0)),
        out_shape=jax.ShapeDtypeStruct((_B, _VP, 1), jnp.float32),
    )(vs_p, msum, cnt, w1_t, b12, w22, b22)


def kernel(clause_indices, variable_states, Wvc, bvc, Wce, bce, Wcv, bcv,
           W1, b1, W2, b2):
    idx4 = clause_indices.transpose(0, 2, 1).reshape(_B, _S, _CB, _CBLK)
    idx4 = idx4.transpose(0, 2, 1, 3)
    vs_p = jnp.pad(variable_states, ((0, 0), (0, _VP - _V), (0, 0)))
    wvc_t, wce_t, wcv_t, w1_t = Wvc.T, Wce.T, Wcv.T, W1.T
    bvc2, bce2, bcv2, b12 = (x.reshape(1, _H) for x in (bvc, bce, bcv, b1))
    w22 = W2.reshape(1, _H)
    b22 = b2.reshape(1, 1)

    for _ in range(_ITERS - 1):
        msum, cnt = _iter_call(idx4, vs_p, wvc_t, bvc2, wce_t, bce2, wcv_t, bcv2)
        vs_p = _update_call(vs_p, msum, cnt)
    msum, cnt = _iter_call(idx4, vs_p, wvc_t, bvc2, wce_t, bce2, wcv_t, bcv2)
    probs = _head_call(vs_p, msum, cnt, w1_t, b12, w22, b22)
    return probs[:, :_V, 0]
